# Initial kernel scaffold; baseline (speedup 1.0000x reference)
import jax
import jax.numpy as jnp
from jax.experimental import pallas as pl


def kernel(x, embedding_tables):
    raise NotImplementedError("write your pallas kernel here")


# speedup vs baseline: 1.5738x; 1.5738x over previous
import functools

import jax
import jax.numpy as jnp
from jax import lax
from jax.experimental import pallas as pl
from jax.experimental.pallas import tpu as pltpu
from jax.experimental.pallas import tpu_sc as plsc

D = 32
B = 16384 * 26
NC = 2
NS = 16
NW = NC * NS
B_PER_W = B // NW
N_CHUNK = 4
C = B_PER_W // N_CHUNK

_mesh = plsc.VectorSubcoreMesh(core_axis_name="c", subcore_axis_name="s")


@functools.partial(
    pl.kernel,
    mesh=_mesh,
    out_type=jax.ShapeDtypeStruct((B, D), jnp.float32),
    compiler_params=pltpu.CompilerParams(use_tc_tiling_on_sc=False),
    scratch_types=[
        pltpu.VMEM((C,), jnp.int32),
        pltpu.VMEM((C, D), jnp.float32),
        pltpu.SemaphoreType.DMA,
    ],
)
def _emb_gather(x_hbm, table_hbm, out_hbm, idx_v, rows_v, sem):
    wid = lax.axis_index("s") * NC + lax.axis_index("c")
    base = wid * B_PER_W

    def body(i, carry):
        off = base + i * C
        pltpu.sync_copy(x_hbm.at[pl.ds(off, C)], idx_v)
        pltpu.async_copy(table_hbm.at[idx_v], rows_v, sem).wait()
        pltpu.sync_copy(rows_v, out_hbm.at[pl.ds(off, C)])
        return carry

    lax.fori_loop(0, N_CHUNK, body, 0)


def kernel(x, embedding_tables):
    x_flat = x.reshape(-1).astype(jnp.int32)
    out = _emb_gather(x_flat, embedding_tables)
    return out.reshape(x.shape + (D,))


# speedup vs baseline: 1.5818x; 1.0051x over previous
import functools

import jax
import jax.numpy as jnp
from jax import lax
from jax.experimental import pallas as pl
from jax.experimental.pallas import tpu as pltpu
from jax.experimental.pallas import tpu_sc as plsc

D = 32
B = 16384 * 26
NC = 2
NS = 16
NW = NC * NS
B_PER_W = B // NW
N_CHUNK = 8
C = B_PER_W // N_CHUNK

_mesh = plsc.VectorSubcoreMesh(core_axis_name="c", subcore_axis_name="s")


@functools.partial(
    pl.kernel,
    mesh=_mesh,
    out_type=jax.ShapeDtypeStruct((B, D), jnp.float32),
    compiler_params=pltpu.CompilerParams(use_tc_tiling_on_sc=False),
    scratch_types=[
        pltpu.VMEM((B_PER_W,), jnp.int32),
        pltpu.VMEM((C, D), jnp.float32),
        pltpu.VMEM((C, D), jnp.float32),
        pltpu.SemaphoreType.DMA,
        pltpu.SemaphoreType.DMA,
        pltpu.SemaphoreType.DMA,
        pltpu.SemaphoreType.DMA,
    ],
)
def _emb_gather(x_hbm, table_hbm, out_hbm, idx_v, rows0, rows1,
                gsem0, gsem1, wsem0, wsem1):
    wid = lax.axis_index("s") * NC + lax.axis_index("c")
    base = wid * B_PER_W

    pltpu.sync_copy(x_hbm.at[pl.ds(base, B_PER_W)], idx_v)

    bufs = (rows0, rows1)
    gsems = (gsem0, gsem1)
    wsems = (wsem0, wsem1)

    gcp = [None] * N_CHUNK
    wcp = [None] * N_CHUNK
    for i in range(N_CHUNK):
        b = i % 2
        if i >= 2:
            wcp[i - 2].wait()
        gcp[i] = pltpu.async_copy(
            table_hbm.at[idx_v.at[pl.ds(i * C, C)]], bufs[b], gsems[b])
        if i >= 1:
            pb = (i - 1) % 2
            gcp[i - 1].wait()
            wcp[i - 1] = pltpu.async_copy(
                bufs[pb], out_hbm.at[pl.ds(base + (i - 1) * C, C)], wsems[pb])

    last = N_CHUNK - 1
    lb = last % 2
    gcp[last].wait()
    wcp[last] = pltpu.async_copy(
        bufs[lb], out_hbm.at[pl.ds(base + last * C, C)], wsems[lb])
    wcp[last - 1].wait()
    wcp[last].wait()


def kernel(x, embedding_tables):
    x_flat = x.reshape(-1).astype(jnp.int32)
    out = _emb_gather(x_flat, embedding_tables)
    return out.reshape(x.shape + (D,))
